# baseline (speedup 1.0000x reference)
import jax
import jax.numpy as jnp
from jax import lax
from jax.experimental import pallas as pl
from jax.experimental.pallas import tpu as pltpu

_N = 16384
_C = 100
_S = 30.0
_BT = 2048
_NB = _N // _BT


def _body(xt_ref, t_ref, ml_ref, out_ref):
    i = pl.program_id(0)
    xt = xt_ref[...]
    t = t_ref[...]
    ml = ml_ref[...]
    row = lax.broadcasted_iota(jnp.int32, (_C, _BT), 0)
    onehot = row == t
    logits = xt * _S - jnp.where(onehot, ml * _S, 0.0)
    m = jnp.max(logits, axis=0, keepdims=True)
    se = jnp.sum(jnp.exp(logits - m), axis=0, keepdims=True)
    tgt = jnp.sum(jnp.where(onehot, logits, 0.0), axis=0, keepdims=True)
    part = jnp.sum(m + jnp.log(se) - tgt)

    @pl.when(i == 0)
    def _():
        out_ref[0, 0] = 0.0

    out_ref[0, 0] += part

    @pl.when(i == _NB - 1)
    def _():
        out_ref[0, 0] = out_ref[0, 0] / _N


def kernel(x, target, m_list):
    out = pl.pallas_call(
        _body,
        grid=(_NB,),
        in_specs=[
            pl.BlockSpec((_C, _BT), lambda i: (0, i)),
            pl.BlockSpec((1, _BT), lambda i: (0, i)),
            pl.BlockSpec((_C, 1), lambda i: (0, 0)),
        ],
        out_specs=pl.BlockSpec(memory_space=pltpu.SMEM),
        out_shape=jax.ShapeDtypeStruct((1, 1), jnp.float32),
        compiler_params=pltpu.CompilerParams(
            dimension_semantics=("arbitrary",),
        ),
    )(x.T, target.reshape(1, _N), m_list.reshape(_C, 1))
    return out[0, 0]


# speedup vs baseline: 1.1486x; 1.1486x over previous
import jax
import jax.numpy as jnp
from jax import lax
from jax.experimental import pallas as pl
from jax.experimental.pallas import tpu as pltpu

_N = 16384
_C = 100
_S = 30.0
_BT = 8192
_NB = _N // _BT


def _body(xt_ref, t_ref, ml_ref, out_ref):
    i = pl.program_id(0)
    xt = xt_ref[...]
    t = t_ref[...]
    ml = ml_ref[...]
    row = lax.broadcasted_iota(jnp.int32, (_C, _BT), 0)
    onehot = row == t
    logits = xt * _S - jnp.where(onehot, ml * _S, 0.0)
    m = jnp.max(logits, axis=0, keepdims=True)
    se = jnp.sum(jnp.exp(logits - m), axis=0, keepdims=True)
    tgt = jnp.sum(jnp.where(onehot, logits, 0.0), axis=0, keepdims=True)
    part = jnp.sum(m + jnp.log(se) - tgt)

    @pl.when(i == 0)
    def _():
        out_ref[0, 0] = 0.0

    out_ref[0, 0] += part

    @pl.when(i == _NB - 1)
    def _():
        out_ref[0, 0] = out_ref[0, 0] / _N


def kernel(x, target, m_list):
    out = pl.pallas_call(
        _body,
        grid=(_NB,),
        in_specs=[
            pl.BlockSpec((_C, _BT), lambda i: (0, i)),
            pl.BlockSpec((1, _BT), lambda i: (0, i)),
            pl.BlockSpec((_C, 1), lambda i: (0, 0)),
        ],
        out_specs=pl.BlockSpec(memory_space=pltpu.SMEM),
        out_shape=jax.ShapeDtypeStruct((1, 1), jnp.float32),
        compiler_params=pltpu.CompilerParams(
            dimension_semantics=("arbitrary",),
        ),
    )(x.T, target.reshape(1, _N), m_list.reshape(_C, 1))
    return out[0, 0]


# speedup vs baseline: 1.1858x; 1.0324x over previous
import jax
import jax.numpy as jnp
from jax import lax
from jax.experimental import pallas as pl
from jax.experimental.pallas import tpu as pltpu

_N = 16384
_C = 100
_S = 30.0
_BT = 4096
_NB = _N // _BT


def _body(xt_ref, t_ref, ml_ref, out_ref):
    i = pl.program_id(0)
    xt = xt_ref[...]
    t = t_ref[...]
    ml = ml_ref[...]
    row = lax.broadcasted_iota(jnp.int32, (_C, _BT), 0)
    onehot = row == t
    logits = xt * _S - jnp.where(onehot, ml * _S, 0.0)
    m = jnp.max(logits, axis=0, keepdims=True)
    se = jnp.sum(jnp.exp(logits - m), axis=0, keepdims=True)
    tgt = jnp.sum(jnp.where(onehot, logits, 0.0), axis=0, keepdims=True)
    part = jnp.sum(m + jnp.log(se) - tgt)

    @pl.when(i == 0)
    def _():
        out_ref[0, 0] = 0.0

    out_ref[0, 0] += part

    @pl.when(i == _NB - 1)
    def _():
        out_ref[0, 0] = out_ref[0, 0] / _N


def kernel(x, target, m_list):
    out = pl.pallas_call(
        _body,
        grid=(_NB,),
        in_specs=[
            pl.BlockSpec((_C, _BT), lambda i: (0, i)),
            pl.BlockSpec((1, _BT), lambda i: (0, i)),
            pl.BlockSpec((_C, 1), lambda i: (0, 0)),
        ],
        out_specs=pl.BlockSpec(memory_space=pltpu.SMEM),
        out_shape=jax.ShapeDtypeStruct((1, 1), jnp.float32),
        compiler_params=pltpu.CompilerParams(
            dimension_semantics=("arbitrary",),
        ),
    )(x.T, target.reshape(1, _N), m_list.reshape(_C, 1))
    return out[0, 0]


# speedup vs baseline: 1.4046x; 1.1845x over previous
import jax
import jax.numpy as jnp
from jax import lax
from jax.experimental import pallas as pl
from jax.experimental.pallas import tpu as pltpu

_N = 16384
_C = 100
_S = 30.0
_BT = 4096
_NB = _N // _BT


def _body(xt_ref, t_ref, ml_ref, out_ref):
    i = pl.program_id(0)
    xt = xt_ref[...]
    t = t_ref[...]
    mlr = ml_ref[...]
    ri = lax.broadcasted_iota(jnp.int32, (_C, _C), 0)
    ci = lax.broadcasted_iota(jnp.int32, (_C, _C), 1)
    ml = jnp.sum(
        jnp.where(ri == ci, jnp.broadcast_to(mlr, (_C, _C)), 0.0),
        axis=1,
        keepdims=True,
    )
    row = lax.broadcasted_iota(jnp.int32, (_C, _BT), 0)
    onehot = row == t
    logits = xt * _S - jnp.where(onehot, ml * _S, 0.0)
    m = jnp.max(logits, axis=0, keepdims=True)
    se = jnp.sum(jnp.exp(logits - m), axis=0, keepdims=True)
    tgt = jnp.sum(jnp.where(onehot, logits, 0.0), axis=0, keepdims=True)
    part = jnp.sum(m + jnp.log(se) - tgt)

    @pl.when(i == 0)
    def _():
        out_ref[0, 0] = 0.0

    out_ref[0, 0] += part

    @pl.when(i == _NB - 1)
    def _():
        out_ref[0, 0] = out_ref[0, 0] / _N


def kernel(x, target, m_list):
    out = pl.pallas_call(
        _body,
        grid=(_NB,),
        in_specs=[
            pl.BlockSpec((_C, _BT), lambda i: (0, i)),
            pl.BlockSpec((1, _BT), lambda i: (0, i)),
            pl.BlockSpec((1, _C), lambda i: (0, 0)),
        ],
        out_specs=pl.BlockSpec(memory_space=pltpu.SMEM),
        out_shape=jax.ShapeDtypeStruct((1, 1), jnp.float32),
        compiler_params=pltpu.CompilerParams(
            dimension_semantics=("arbitrary",),
        ),
    )(x.T, target.reshape(1, _N), m_list.reshape(1, _C))
    return out[0, 0]
